# baseline (speedup 1.0000x reference)
import functools

import jax
import jax.numpy as jnp
import numpy as np
from jax import lax
from jax.experimental import pallas as pl
from jax.experimental.pallas import tpu as pltpu
from jax.experimental.pallas import tpu_sc as plsc

N_NODES = 10000
N_EDGES = 160000
D_FEAT = 256

NC = 2
NS = 16
LANES = 16
GROUP = 128
N_GROUPS = 1280
E_PAD = N_GROUPS * GROUP
G_PER_TILE = N_GROUPS // NS
N_PAD = 10240
ROWS_PER_TILE = N_PAD // NS
DH = 128
DI = DH // 2
NBUF = 2
IDXC = 8
N_CHUNK = G_PER_TILE // IDXC

_PERM = np.concatenate([
    np.concatenate([np.arange(32 * k, 32 * k + 32, 2),
                    np.arange(32 * k + 1, 32 * k + 32, 2)])
    for k in range(DH // 32)
])


def _sc_accumulate(xi, src_g, dst_g, zz, zd):
    mesh = plsc.VectorSubcoreMesh(core_axis_name="c", subcore_axis_name="s")

    @functools.partial(
        pl.kernel,
        out_type=(
            jax.ShapeDtypeStruct((NC, N_PAD, DH), jnp.float32),
            jax.ShapeDtypeStruct((NS, N_PAD), jnp.float32),
        ),
        mesh=mesh,
        compiler_params=pltpu.CompilerParams(needs_layout_passes=False,
                                             use_tc_tiling_on_sc=False),
        scratch_types=[
            pltpu.VMEM((2, IDXC, GROUP), jnp.int32),
            pltpu.VMEM((2, IDXC, GROUP), jnp.int32),
            pltpu.VMEM((NBUF, GROUP, DI), jnp.int32),
            pltpu.VMEM((GROUP, DH), jnp.float32),
            pltpu.VMEM((N_PAD,), jnp.float32),
            pltpu.VMEM_SHARED((N_PAD, DH), jnp.float32),
        ] + [pltpu.SemaphoreType.DMA] * (NBUF + 2),
    )
    def k(xi_hbm, src_hbm, dst_hbm, zz_hbm, zd_hbm, out_hbm, outd_hbm,
          src_v, dst_v, rows_v, fbuf, deg_v, acc, *sems):
        gsems, isems = sems[:NBUF], sems[NBUF:]
        c = lax.axis_index("c")
        s = lax.axis_index("s")
        pltpu.sync_copy(zd_hbm, deg_v)
        r0 = s * ROWS_PER_TILE
        pltpu.sync_copy(zz_hbm.at[pl.ds(r0, ROWS_PER_TILE)],
                        acc.at[pl.ds(r0, ROWS_PER_TILE)])

        def idx_load(ci, ib):
            sl = pl.ds(ci * IDXC, IDXC)
            pltpu.async_copy(src_hbm.at[s, sl], src_v.at[ib], isems[ib])
            pltpu.async_copy(dst_hbm.at[s, sl], dst_v.at[ib], isems[ib])

        def idx_wait(ib):
            sl = pl.ds(0, IDXC)
            pltpu.make_async_copy(src_hbm.at[s, sl], src_v.at[ib],
                                  isems[ib]).wait()
            pltpu.make_async_copy(dst_hbm.at[s, sl], dst_v.at[ib],
                                  isems[ib]).wait()

        idx_load(0, 0)
        idx_wait(0)
        plsc.subcore_barrier()
        table = xi_hbm.at[c]
        ones16 = jnp.ones((LANES,), jnp.float32)

        def gather(ib, g, b):
            pltpu.async_copy(table.at[src_v.at[ib, g]], rows_v.at[b],
                             gsems[b])

        def gather_wait(b):
            pltpu.make_async_copy(table.at[src_v.at[0, 0]], rows_v.at[b],
                                  gsems[b]).wait()

        def unpack_rows(b):
            def row_body(r, carry):
                for kk in range(DI // LANES):
                    v = rows_v[b, r, pl.ds(kk * LANES, LANES)]
                    bf = plsc.bitcast(v, jnp.bfloat16)
                    ea, ob = plsc.unpack(
                        bf, format=plsc.PackFormat.INTERLEAVED,
                        preferred_element_type=jnp.float32)
                    fbuf[r, pl.ds(32 * kk, LANES)] = ea
                    fbuf[r, pl.ds(32 * kk + LANES, LANES)] = ob
                return carry
            lax.fori_loop(0, GROUP, row_body, 0)

        def chunk(ci, cp):
            @pl.when(ci < N_CHUNK - 1)
            def _pf():
                idx_load(ci + 1, 1 - cp)

            for b in range(NBUF):
                gather(cp, b, b)
            for g in range(IDXC):
                b = g % NBUF
                gather_wait(b)
                unpack_rows(b)
                pltpu.sync_copy(fbuf, acc.at[dst_v.at[cp, g]], add=True)
                if g + NBUF < IDXC:
                    gather(cp, g + NBUF, b)
                @pl.when(c == 0)
                def _histo():
                    for kk in range(GROUP // LANES):
                        d = dst_v[cp, g, pl.ds(kk * LANES, LANES)]
                        plsc.addupdate_scatter(deg_v, [d], ones16)
            @pl.when(ci < N_CHUNK - 1)
            def _pfw():
                idx_wait(1 - cp)

        def outer(si, carry):
            chunk(2 * si, 0)
            chunk(2 * si + 1, 1)
            return carry

        lax.fori_loop(0, N_CHUNK // 2, outer, 0)
        plsc.subcore_barrier()
        pltpu.sync_copy(acc.at[pl.ds(r0, ROWS_PER_TILE)],
                        out_hbm.at[c, pl.ds(r0, ROWS_PER_TILE)])

        @pl.when(c == 0)
        def _write_deg():
            pltpu.sync_copy(deg_v, outd_hbm.at[s])

    return k(xi, src_g, dst_g, zz, zd)


ROW_BLK = 512


def _tc_transform(a, degs, w0p, w1p, b2):
    def body(a0_ref, a1_ref, deg_ref, w0_ref, w1_ref, b_ref, o_ref):
        acc = jnp.dot(a0_ref[0], w0_ref[...],
                      preferred_element_type=jnp.float32,
                      precision=lax.Precision.HIGHEST)
        acc += jnp.dot(a1_ref[0], w1_ref[...],
                       preferred_element_type=jnp.float32,
                       precision=lax.Precision.HIGHEST)
        deg = jnp.sum(deg_ref[...], axis=0)
        acc += deg[:, None] * b_ref[...]
        o_ref[...] = jnp.tanh(acc)

    return pl.pallas_call(
        body,
        grid=(N_PAD // ROW_BLK,),
        in_specs=[
            pl.BlockSpec((1, ROW_BLK, DH), lambda i: (0, i, 0)),
            pl.BlockSpec((1, ROW_BLK, DH), lambda i: (1, i, 0)),
            pl.BlockSpec((NS, ROW_BLK), lambda i: (0, i)),
            pl.BlockSpec((DH, D_FEAT), lambda i: (0, 0)),
            pl.BlockSpec((DH, D_FEAT), lambda i: (0, 0)),
            pl.BlockSpec((1, D_FEAT), lambda i: (0, 0)),
        ],
        out_specs=pl.BlockSpec((ROW_BLK, D_FEAT), lambda i: (i, 0)),
        out_shape=jax.ShapeDtypeStruct((N_NODES, D_FEAT), jnp.float32),
    )(a, a, degs, w0p, w1p, b2)


def kernel(x, edge_index, W, b):
    src = edge_index[0]
    dst = edge_index[1]
    xb = x.astype(jnp.bfloat16)
    xh = jnp.stack([xb[:, :DH], xb[:, DH:]])
    xi = lax.bitcast_convert_type(
        xh.reshape(NC, N_NODES, DI, 2), jnp.int32)
    npad = E_PAD - N_EDGES
    src_g = jnp.concatenate([src, jnp.zeros((npad,), jnp.int32)])
    dst_g = jnp.concatenate([dst, jnp.full((npad,), N_NODES, jnp.int32)])
    src_g = src_g.reshape(NS, G_PER_TILE, GROUP)
    dst_g = dst_g.reshape(NS, G_PER_TILE, GROUP)
    zz = jnp.zeros((N_PAD, DH), jnp.float32)
    zd = jnp.zeros((N_PAD,), jnp.float32)

    a, degs = _sc_accumulate(xi, src_g, dst_g, zz, zd)
    w0p = W[:DH][_PERM]
    w1p = W[DH:][_PERM]
    return _tc_transform(a, degs, w0p, w1p, b.reshape(1, D_FEAT))


# speedup vs baseline: 1.2234x; 1.2234x over previous
import functools

import jax
import jax.numpy as jnp
from jax import lax
from jax.experimental import pallas as pl
from jax.experimental.pallas import tpu as pltpu
from jax.experimental.pallas import tpu_sc as plsc

N_NODES = 10000
N_EDGES = 160000
D_FEAT = 256

NC = 2
NS = 16
LANES = 16
GROUP = 128
N_GROUPS = 1280
E_PAD = N_GROUPS * GROUP
G_PER_TILE = N_GROUPS // NS
N_PAD = 10240
ROWS_PER_TILE = N_PAD // NS
DH = 128
NBUF = 2
IDXC = 8
N_CHUNK = G_PER_TILE // IDXC


def _sc_accumulate(xa, src_g, dst_g, zz, zd):
    mesh = plsc.VectorSubcoreMesh(core_axis_name="c", subcore_axis_name="s")

    @functools.partial(
        pl.kernel,
        out_type=(
            jax.ShapeDtypeStruct((NC, N_PAD, DH), jnp.float32),
            jax.ShapeDtypeStruct((NC, NS, N_PAD), jnp.float32),
        ),
        mesh=mesh,
        compiler_params=pltpu.CompilerParams(needs_layout_passes=False),
        scratch_types=[
            pltpu.VMEM((2, IDXC, GROUP), jnp.int32),
            pltpu.VMEM((2, IDXC, GROUP), jnp.int32),
            pltpu.VMEM((NBUF, GROUP, DH), jnp.float32),
            pltpu.VMEM((N_PAD,), jnp.float32),
            pltpu.VMEM_SHARED((N_PAD, DH), jnp.float32),
        ] + [pltpu.SemaphoreType.DMA] * (NBUF + 2),
    )
    def k(xa_hbm, src_hbm, dst_hbm, zz_hbm, zd_hbm, out_hbm, outd_hbm,
          src_v, dst_v, rows_v, deg_v, acc, *sems):
        gsems, isems = sems[:NBUF], sems[NBUF:]
        c = lax.axis_index("c")
        s = lax.axis_index("s")
        pltpu.sync_copy(zd_hbm, deg_v)
        r0 = s * ROWS_PER_TILE
        pltpu.sync_copy(zz_hbm.at[pl.ds(r0, ROWS_PER_TILE)],
                        acc.at[pl.ds(r0, ROWS_PER_TILE)])

        def idx_load(ci, ib):
            sl = pl.ds(ci * IDXC, IDXC)
            pltpu.async_copy(src_hbm.at[s, sl], src_v.at[ib], isems[ib])
            pltpu.async_copy(dst_hbm.at[s, sl], dst_v.at[ib], isems[ib])

        def idx_wait(ib):
            sl = pl.ds(0, IDXC)
            pltpu.make_async_copy(src_hbm.at[s, sl], src_v.at[ib],
                                  isems[ib]).wait()
            pltpu.make_async_copy(dst_hbm.at[s, sl], dst_v.at[ib],
                                  isems[ib]).wait()

        idx_load(0, 0)
        idx_wait(0)
        plsc.subcore_barrier()
        table = xa_hbm.at[c]
        ones16 = jnp.ones((LANES,), jnp.float32)

        def gather(ib, g, b):
            pltpu.async_copy(table.at[src_v.at[ib, g]], rows_v.at[b],
                             gsems[b])

        def gather_wait(b):
            pltpu.make_async_copy(table.at[src_v.at[0, 0]], rows_v.at[b],
                                  gsems[b]).wait()

        def chunk(ci, cp):
            @pl.when(ci < N_CHUNK - 1)
            def _pf():
                idx_load(ci + 1, 1 - cp)

            for b in range(NBUF):
                gather(cp, b, b)
            for g in range(IDXC):
                b = g % NBUF
                gather_wait(b)
                pltpu.sync_copy(rows_v.at[b], acc.at[dst_v.at[cp, g]],
                                add=True)
                if g + NBUF < IDXC:
                    gather(cp, g + NBUF, b)
                @pl.when(c == cp)
                def _histo():
                    for kk in range(GROUP // LANES):
                        d = dst_v[cp, g, pl.ds(kk * LANES, LANES)]
                        plsc.addupdate_scatter(deg_v, [d], ones16)
            @pl.when(ci < N_CHUNK - 1)
            def _pfw():
                idx_wait(1 - cp)

        def outer(si, carry):
            chunk(2 * si, 0)
            chunk(2 * si + 1, 1)
            return carry

        lax.fori_loop(0, N_CHUNK // 2, outer, 0)
        plsc.subcore_barrier()
        pltpu.sync_copy(acc.at[pl.ds(r0, ROWS_PER_TILE)],
                        out_hbm.at[c, pl.ds(r0, ROWS_PER_TILE)])

        pltpu.sync_copy(deg_v, outd_hbm.at[c, s])

    return k(xa, src_g, dst_g, zz, zd)


ROW_BLK = 512


def _tc_transform(a, degs, W, b2):
    def body(a0_ref, a1_ref, deg_ref, w_ref, b_ref, o_ref):
        a0 = a0_ref[0]
        a1 = a1_ref[0]
        w = w_ref[...]
        acc = jnp.dot(a0, w[:DH, :], preferred_element_type=jnp.float32,
                      precision=lax.Precision.HIGHEST)
        acc += jnp.dot(a1, w[DH:, :], preferred_element_type=jnp.float32,
                       precision=lax.Precision.HIGHEST)
        deg = jnp.sum(deg_ref[...], axis=0)
        acc += deg[:, None] * b_ref[...]
        o_ref[...] = jnp.tanh(acc)

    return pl.pallas_call(
        body,
        grid=(N_PAD // ROW_BLK,),
        in_specs=[
            pl.BlockSpec((1, ROW_BLK, DH), lambda i: (0, i, 0)),
            pl.BlockSpec((1, ROW_BLK, DH), lambda i: (1, i, 0)),
            pl.BlockSpec((NC * NS, ROW_BLK), lambda i: (0, i)),
            pl.BlockSpec((D_FEAT, D_FEAT), lambda i: (0, 0)),
            pl.BlockSpec((1, D_FEAT), lambda i: (0, 0)),
        ],
        out_specs=pl.BlockSpec((ROW_BLK, D_FEAT), lambda i: (i, 0)),
        out_shape=jax.ShapeDtypeStruct((N_NODES, D_FEAT), jnp.float32),
    )(a, a, degs, W, b2)


def kernel(x, edge_index, W, b):
    src = edge_index[0]
    dst = edge_index[1]
    xa = jnp.stack([x[:, :DH], x[:, DH:]])
    npad = E_PAD - N_EDGES
    src_g = jnp.concatenate([src, jnp.zeros((npad,), jnp.int32)])
    dst_g = jnp.concatenate([dst, jnp.full((npad,), N_NODES, jnp.int32)])
    src_g = src_g.reshape(NS, G_PER_TILE, GROUP)
    dst_g = dst_g.reshape(NS, G_PER_TILE, GROUP)
    zz = jnp.zeros((N_PAD, DH), jnp.float32)
    zd = jnp.zeros((N_PAD,), jnp.float32)

    a, degs = _sc_accumulate(xa, src_g, dst_g, zz, zd)
    degs2 = degs.reshape(NC * NS, N_PAD)
    return _tc_transform(a, degs2, W, b.reshape(1, D_FEAT))


# speedup vs baseline: 1.2645x; 1.0336x over previous
import functools

import jax
import jax.numpy as jnp
from jax import lax
from jax.experimental import pallas as pl
from jax.experimental.pallas import tpu as pltpu
from jax.experimental.pallas import tpu_sc as plsc

N_NODES = 10000
N_EDGES = 160000
D_FEAT = 256

NC = 2
NS = 16
LANES = 16
GROUP = 128
N_GROUPS = 1280
E_PAD = N_GROUPS * GROUP
G_PER_TILE = N_GROUPS // NS
N_PAD = 10240
ROWS_PER_TILE = N_PAD // NS
DH = 128
NBUF = 2
IDXC = 8
N_CHUNK = G_PER_TILE // IDXC


def _sc_accumulate(xa, src_g, dst_g, zz, zd):
    mesh = plsc.VectorSubcoreMesh(core_axis_name="c", subcore_axis_name="s")

    @functools.partial(
        pl.kernel,
        out_type=(
            jax.ShapeDtypeStruct((NC, N_PAD, DH), jnp.float32),
            jax.ShapeDtypeStruct((NC, NS, N_PAD), jnp.float32),
        ),
        mesh=mesh,
        compiler_params=pltpu.CompilerParams(needs_layout_passes=False,
                                             use_tc_tiling_on_sc=False),
        scratch_types=[
            pltpu.VMEM((2, IDXC, GROUP), jnp.int32),
            pltpu.VMEM((2, IDXC, GROUP), jnp.int32),
            pltpu.VMEM((NBUF, GROUP, DH), jnp.float32),
            pltpu.VMEM((N_PAD,), jnp.float32),
            pltpu.VMEM_SHARED((N_PAD, DH), jnp.float32),
        ] + [pltpu.SemaphoreType.DMA] * (NBUF + 2),
    )
    def k(xa_hbm, src_hbm, dst_hbm, zz_hbm, zd_hbm, out_hbm, outd_hbm,
          src_v, dst_v, rows_v, deg_v, acc, *sems):
        gsems, isems = sems[:NBUF], sems[NBUF:]
        c = lax.axis_index("c")
        s = lax.axis_index("s")
        pltpu.sync_copy(zd_hbm, deg_v)
        r0 = s * ROWS_PER_TILE
        pltpu.sync_copy(zz_hbm.at[pl.ds(r0, ROWS_PER_TILE)],
                        acc.at[pl.ds(r0, ROWS_PER_TILE)])

        def idx_load(ci, ib):
            sl = pl.ds(ci * IDXC, IDXC)
            pltpu.async_copy(src_hbm.at[s, sl], src_v.at[ib], isems[ib])
            pltpu.async_copy(dst_hbm.at[s, sl], dst_v.at[ib], isems[ib])

        def idx_wait(ib):
            sl = pl.ds(0, IDXC)
            pltpu.make_async_copy(src_hbm.at[s, sl], src_v.at[ib],
                                  isems[ib]).wait()
            pltpu.make_async_copy(dst_hbm.at[s, sl], dst_v.at[ib],
                                  isems[ib]).wait()

        idx_load(0, 0)
        idx_wait(0)
        plsc.subcore_barrier()
        table = xa_hbm.at[c]
        ones16 = jnp.ones((LANES,), jnp.float32)

        def gather(ib, g, b):
            pltpu.async_copy(table.at[src_v.at[ib, g]], rows_v.at[b],
                             gsems[b])

        def gather_wait(b):
            pltpu.make_async_copy(table.at[src_v.at[0, 0]], rows_v.at[b],
                                  gsems[b]).wait()

        def chunk(ci, cp):
            @pl.when(ci < N_CHUNK - 1)
            def _pf():
                idx_load(ci + 1, 1 - cp)

            for b in range(NBUF):
                gather(cp, b, b)
            for g in range(IDXC):
                b = g % NBUF
                gather_wait(b)
                pltpu.sync_copy(rows_v.at[b], acc.at[dst_v.at[cp, g]],
                                add=True)
                if g + NBUF < IDXC:
                    gather(cp, g + NBUF, b)
                @pl.when(c == cp)
                def _histo():
                    for kk in range(GROUP // LANES):
                        d = dst_v[cp, g, pl.ds(kk * LANES, LANES)]
                        plsc.addupdate_scatter(deg_v, [d], ones16)
            @pl.when(ci < N_CHUNK - 1)
            def _pfw():
                idx_wait(1 - cp)

        def outer(si, carry):
            chunk(2 * si, 0)
            chunk(2 * si + 1, 1)
            return carry

        lax.fori_loop(0, N_CHUNK // 2, outer, 0)
        plsc.subcore_barrier()
        pltpu.sync_copy(acc.at[pl.ds(r0, ROWS_PER_TILE)],
                        out_hbm.at[c, pl.ds(r0, ROWS_PER_TILE)])

        pltpu.sync_copy(deg_v, outd_hbm.at[c, s])

    return k(xa, src_g, dst_g, zz, zd)


ROW_BLK = 512


def _tc_transform(a, degs, W, b2):
    def body(a0_ref, a1_ref, deg_ref, w_ref, b_ref, o_ref):
        a0 = a0_ref[0]
        a1 = a1_ref[0]
        w = w_ref[...]
        acc = jnp.dot(a0, w[:DH, :], preferred_element_type=jnp.float32,
                      precision=lax.Precision.HIGHEST)
        acc += jnp.dot(a1, w[DH:, :], preferred_element_type=jnp.float32,
                       precision=lax.Precision.HIGHEST)
        deg = jnp.sum(deg_ref[...], axis=0)
        acc += deg[:, None] * b_ref[...]
        o_ref[...] = jnp.tanh(acc)

    return pl.pallas_call(
        body,
        grid=(N_PAD // ROW_BLK,),
        in_specs=[
            pl.BlockSpec((1, ROW_BLK, DH), lambda i: (0, i, 0)),
            pl.BlockSpec((1, ROW_BLK, DH), lambda i: (1, i, 0)),
            pl.BlockSpec((NC * NS, ROW_BLK), lambda i: (0, i)),
            pl.BlockSpec((D_FEAT, D_FEAT), lambda i: (0, 0)),
            pl.BlockSpec((1, D_FEAT), lambda i: (0, 0)),
        ],
        out_specs=pl.BlockSpec((ROW_BLK, D_FEAT), lambda i: (i, 0)),
        out_shape=jax.ShapeDtypeStruct((N_NODES, D_FEAT), jnp.float32),
    )(a, a, degs, W, b2)


def kernel(x, edge_index, W, b):
    src = edge_index[0]
    dst = edge_index[1]
    xa = jnp.stack([x[:, :DH], x[:, DH:]])
    npad = E_PAD - N_EDGES
    src_g = jnp.concatenate([src, jnp.zeros((npad,), jnp.int32)])
    dst_g = jnp.concatenate([dst, jnp.full((npad,), N_NODES, jnp.int32)])
    src_g = src_g.reshape(NS, G_PER_TILE, GROUP)
    dst_g = dst_g.reshape(NS, G_PER_TILE, GROUP)
    zz = jnp.zeros((N_PAD, DH), jnp.float32)
    zd = jnp.zeros((N_PAD,), jnp.float32)

    a, degs = _sc_accumulate(xa, src_g, dst_g, zz, zd)
    degs2 = degs.reshape(NC * NS, N_PAD)
    return _tc_transform(a, degs2, W, b.reshape(1, D_FEAT))
